# baseline (speedup 1.0000x reference)
import jax
import jax.numpy as jnp
from jax.experimental import pallas as pl
from jax.experimental.pallas import tpu as pltpu

B = 8
V = 1_000_000
BLK_W = 32_768
N_BLK = (V + BLK_W - 1) // BLK_W
CHUNK_W = 512
CH_PER_BLK = BLK_W // CHUNK_W
N_CHUNKS = N_BLK * CH_PER_BLK
NEG_INF = float("-inf")
BIG_I32 = 2**31 - 1


def _phase_a_body(x_ref, beam_ref, rows_ref, cols_ref, osel_ref, bflat_ref,
                  bcol_ref, cmax_ref, m_ref, s_ref):
    i = pl.program_id(0)

    @pl.when(i == 0)
    def _init():
        m_ref[...] = jnp.full((B, 1), NEG_INF, jnp.float32)
        s_ref[...] = jnp.zeros((B, 1), jnp.float32)

    x = x_ref[...]
    col = i * BLK_W + jax.lax.broadcasted_iota(jnp.int32, (B, BLK_W), 1)
    x = jnp.where(col < V, x, NEG_INF)

    tile = jnp.max(x.reshape(B, CH_PER_BLK, CHUNK_W), axis=2)
    cmax_ref[pl.ds(i, 1)] = tile[None]

    bm = jnp.max(tile, axis=1, keepdims=True)
    m_old = m_ref[...]
    m_new = jnp.maximum(m_old, bm)
    s_contrib = jnp.sum(jnp.exp(x - m_new), axis=1, keepdims=True)
    s_ref[...] = s_ref[...] * jnp.exp(m_old - m_new) + s_contrib
    m_ref[...] = m_new

    @pl.when(i == N_BLK - 1)
    def _fin():
        lse = m_ref[...] + jnp.log(s_ref[...])
        o = beam_ref[...] - lse
        biased = cmax_ref[...] + o[None]
        sh = (N_BLK, B, CH_PER_BLK)
        d0 = jax.lax.broadcasted_iota(jnp.int32, sh, 0)
        d1 = jax.lax.broadcasted_iota(jnp.int32, sh, 1)
        d2 = jax.lax.broadcasted_iota(jnp.int32, sh, 2)
        key = d1 * N_CHUNKS + d0 * CH_PER_BLK + d2

        def _amax(v):
            return jnp.max(
                jnp.max(jnp.max(v, axis=2, keepdims=True), axis=1,
                        keepdims=True), axis=0, keepdims=True)

        def _amin(v):
            return jnp.min(
                jnp.min(jnp.min(v, axis=2, keepdims=True), axis=1,
                        keepdims=True), axis=0, keepdims=True)

        for k in range(B):
            m = _amax(biased)
            sel = _amin(jnp.where(biased == m, key, BIG_I32))
            r = sel // N_CHUNKS
            c = sel % N_CHUNKS
            rows_ref[0:1, k : k + 1] = r[0]
            cols_ref[0:1, k : k + 1] = c[0]
            osel_ref[0:1, k : k + 1] = _amax(
                jnp.where(d1 == r, o[None], NEG_INF))[0]
            bflat_ref[0:1, k : k + 1] = (r * V + c * CHUNK_W)[0]
            bcol_ref[0:1, k : k + 1] = (c * CHUNK_W)[0]
            biased = jnp.where(key == sel, NEG_INF, biased)


def _phase_b_body(rows_sref, cols_sref, blk_ref, osel_ref, bflat_ref,
                  bcol_ref, scores_ref, prev_ref, tok_ref, gath_ref):
    i = pl.program_id(0)
    r = rows_sref[i]
    gath_ref[pl.ds(i, 1), :] = blk_ref[pl.ds(r, 1), :]

    @pl.when(i == B - 1)
    def _fin():
        lane = jax.lax.broadcasted_iota(jnp.int32, (B, CHUNK_W), 1)
        colg = bcol_ref[...] + lane
        cand = jnp.where(colg < V, gath_ref[...] + osel_ref[...], NEG_INF)
        flat = bflat_ref[...] + lane

        def _amax(x):
            return jnp.max(
                jnp.max(x, axis=1, keepdims=True), axis=0, keepdims=True
            )

        def _amin(x):
            return jnp.min(
                jnp.min(x, axis=1, keepdims=True), axis=0, keepdims=True
            )

        for k in range(B):
            m = _amax(cand)
            sel = _amin(jnp.where(cand == m, flat, BIG_I32))
            scores_ref[0:1, k : k + 1] = m
            prev_ref[0:1, k : k + 1] = sel // V
            tok_ref[0:1, k : k + 1] = sel % V
            cand = jnp.where(flat == sel, NEG_INF, cand)


def kernel(logits_last, beam_scores):
    rows, cols, osel, bflat, bcol = pl.pallas_call(
        _phase_a_body,
        grid=(N_BLK,),
        in_specs=[
            pl.BlockSpec((B, BLK_W), lambda i: (0, i)),
            pl.BlockSpec((B, 1), lambda i: (0, 0)),
        ],
        out_specs=[
            pl.BlockSpec((1, B), lambda i: (0, 0)),
            pl.BlockSpec((1, B), lambda i: (0, 0)),
            pl.BlockSpec((1, B), lambda i: (0, 0)),
            pl.BlockSpec((1, B), lambda i: (0, 0)),
            pl.BlockSpec((1, B), lambda i: (0, 0)),
        ],
        out_shape=[
            jax.ShapeDtypeStruct((1, B), jnp.int32),
            jax.ShapeDtypeStruct((1, B), jnp.int32),
            jax.ShapeDtypeStruct((1, B), jnp.float32),
            jax.ShapeDtypeStruct((1, B), jnp.int32),
            jax.ShapeDtypeStruct((1, B), jnp.int32),
        ],
        scratch_shapes=[
            pltpu.VMEM((N_BLK, B, CH_PER_BLK), jnp.float32),
            pltpu.VMEM((B, 1), jnp.float32),
            pltpu.VMEM((B, 1), jnp.float32),
        ],
    )(logits_last, beam_scores.reshape(B, 1))

    scores, prev, tok = pl.pallas_call(
        _phase_b_body,
        grid_spec=pltpu.PrefetchScalarGridSpec(
            num_scalar_prefetch=2,
            grid=(B,),
            in_specs=[
                pl.BlockSpec(
                    (B, CHUNK_W), lambda i, rows, cols: (0, cols[i])
                ),
                pl.BlockSpec((B, 1), lambda i, rows, cols: (0, 0)),
                pl.BlockSpec((B, 1), lambda i, rows, cols: (0, 0)),
                pl.BlockSpec((B, 1), lambda i, rows, cols: (0, 0)),
            ],
            out_specs=[
                pl.BlockSpec((1, B), lambda i, rows, cols: (0, 0)),
                pl.BlockSpec((1, B), lambda i, rows, cols: (0, 0)),
                pl.BlockSpec((1, B), lambda i, rows, cols: (0, 0)),
            ],
            scratch_shapes=[pltpu.VMEM((B, CHUNK_W), jnp.float32)],
        ),
        out_shape=[
            jax.ShapeDtypeStruct((1, B), jnp.float32),
            jax.ShapeDtypeStruct((1, B), jnp.int32),
            jax.ShapeDtypeStruct((1, B), jnp.int32),
        ],
    )(
        rows.reshape(B),
        cols.reshape(B),
        logits_last,
        osel.reshape(B, 1),
        bflat.reshape(B, 1),
        bcol.reshape(B, 1),
    )

    return scores.reshape(B), prev.reshape(B), tok.reshape(B)


# speedup vs baseline: 1.0449x; 1.0449x over previous
import jax
import jax.numpy as jnp
from jax.experimental import pallas as pl
from jax.experimental.pallas import tpu as pltpu

B = 8
V = 1_000_000
BLK_W = 32_768
N_BLK = (V + BLK_W - 1) // BLK_W
CHUNK_W = 512
CH_PER_BLK = BLK_W // CHUNK_W
N_CHUNKS = N_BLK * CH_PER_BLK
NEG_INF = float("-inf")
BIG_I32 = 2**31 - 1


def _phase_a_body(x_ref, beam_ref, rows_ref, cols_ref, osel_ref, bflat_ref,
                  bcol_ref, cmax_ref, m_ref, s_ref):
    i = pl.program_id(0)

    @pl.when(i == 0)
    def _init():
        m_ref[...] = jnp.full((B, 1), NEG_INF, jnp.float32)
        s_ref[...] = jnp.zeros((B, 1), jnp.float32)

    def _update(x):
        tile = jnp.max(x.reshape(B, CH_PER_BLK, CHUNK_W), axis=2)
        cmax_ref[pl.ds(i, 1)] = tile[None]

        bm = jnp.max(tile, axis=1, keepdims=True)
        m_old = m_ref[...]
        m_new = jnp.maximum(m_old, bm)
        s_contrib = jnp.sum(jnp.exp(x - m_new), axis=1, keepdims=True)
        s_ref[...] = s_ref[...] * jnp.exp(m_old - m_new) + s_contrib
        m_ref[...] = m_new

    @pl.when(i < N_BLK - 1)
    def _fast():
        _update(x_ref[...])

    @pl.when(i == N_BLK - 1)
    def _fin():
        col = i * BLK_W + jax.lax.broadcasted_iota(jnp.int32, (B, BLK_W), 1)
        _update(jnp.where(col < V, x_ref[...], NEG_INF))

        lse = m_ref[...] + jnp.log(s_ref[...])
        o = beam_ref[...] - lse
        biased = cmax_ref[...] + o[None]
        sh = (N_BLK, B, CH_PER_BLK)
        d0 = jax.lax.broadcasted_iota(jnp.int32, sh, 0)
        d1 = jax.lax.broadcasted_iota(jnp.int32, sh, 1)
        d2 = jax.lax.broadcasted_iota(jnp.int32, sh, 2)
        key = d1 * N_CHUNKS + d0 * CH_PER_BLK + d2

        def _amax(v):
            return jnp.max(
                jnp.max(jnp.max(v, axis=2, keepdims=True), axis=1,
                        keepdims=True), axis=0, keepdims=True)

        def _amin(v):
            return jnp.min(
                jnp.min(jnp.min(v, axis=2, keepdims=True), axis=1,
                        keepdims=True), axis=0, keepdims=True)

        for k in range(B):
            m = _amax(biased)
            sel = _amin(jnp.where(biased == m, key, BIG_I32))
            r = sel // N_CHUNKS
            c = sel % N_CHUNKS
            rows_ref[0:1, k : k + 1] = r[0]
            cols_ref[0:1, k : k + 1] = c[0]
            osel_ref[0:1, k : k + 1] = _amax(
                jnp.where(d1 == r, o[None], NEG_INF))[0]
            bflat_ref[0:1, k : k + 1] = (r * V + c * CHUNK_W)[0]
            bcol_ref[0:1, k : k + 1] = (c * CHUNK_W)[0]
            biased = jnp.where(key == sel, NEG_INF, biased)


def _phase_b_body(rows_sref, cols_sref, blk_ref, osel_ref, bflat_ref,
                  bcol_ref, scores_ref, prev_ref, tok_ref, gath_ref):
    i = pl.program_id(0)
    r = rows_sref[i]
    gath_ref[pl.ds(i, 1), :] = blk_ref[pl.ds(r, 1), :]

    @pl.when(i == B - 1)
    def _fin():
        lane = jax.lax.broadcasted_iota(jnp.int32, (B, CHUNK_W), 1)
        colg = bcol_ref[...] + lane
        cand = jnp.where(colg < V, gath_ref[...] + osel_ref[...], NEG_INF)
        flat = bflat_ref[...] + lane

        def _amax(x):
            return jnp.max(
                jnp.max(x, axis=1, keepdims=True), axis=0, keepdims=True
            )

        def _amin(x):
            return jnp.min(
                jnp.min(x, axis=1, keepdims=True), axis=0, keepdims=True
            )

        for k in range(B):
            m = _amax(cand)
            sel = _amin(jnp.where(cand == m, flat, BIG_I32))
            scores_ref[0:1, k : k + 1] = m
            prev_ref[0:1, k : k + 1] = sel // V
            tok_ref[0:1, k : k + 1] = sel % V
            cand = jnp.where(flat == sel, NEG_INF, cand)


def kernel(logits_last, beam_scores):
    rows, cols, osel, bflat, bcol = pl.pallas_call(
        _phase_a_body,
        grid=(N_BLK,),
        in_specs=[
            pl.BlockSpec((B, BLK_W), lambda i: (0, i)),
            pl.BlockSpec((B, 1), lambda i: (0, 0)),
        ],
        out_specs=[
            pl.BlockSpec((1, B), lambda i: (0, 0)),
            pl.BlockSpec((1, B), lambda i: (0, 0)),
            pl.BlockSpec((1, B), lambda i: (0, 0)),
            pl.BlockSpec((1, B), lambda i: (0, 0)),
            pl.BlockSpec((1, B), lambda i: (0, 0)),
        ],
        out_shape=[
            jax.ShapeDtypeStruct((1, B), jnp.int32),
            jax.ShapeDtypeStruct((1, B), jnp.int32),
            jax.ShapeDtypeStruct((1, B), jnp.float32),
            jax.ShapeDtypeStruct((1, B), jnp.int32),
            jax.ShapeDtypeStruct((1, B), jnp.int32),
        ],
        scratch_shapes=[
            pltpu.VMEM((N_BLK, B, CH_PER_BLK), jnp.float32),
            pltpu.VMEM((B, 1), jnp.float32),
            pltpu.VMEM((B, 1), jnp.float32),
        ],
    )(logits_last, beam_scores.reshape(B, 1))

    scores, prev, tok = pl.pallas_call(
        _phase_b_body,
        grid_spec=pltpu.PrefetchScalarGridSpec(
            num_scalar_prefetch=2,
            grid=(B,),
            in_specs=[
                pl.BlockSpec(
                    (B, CHUNK_W), lambda i, rows, cols: (0, cols[i])
                ),
                pl.BlockSpec((B, 1), lambda i, rows, cols: (0, 0)),
                pl.BlockSpec((B, 1), lambda i, rows, cols: (0, 0)),
                pl.BlockSpec((B, 1), lambda i, rows, cols: (0, 0)),
            ],
            out_specs=[
                pl.BlockSpec((1, B), lambda i, rows, cols: (0, 0)),
                pl.BlockSpec((1, B), lambda i, rows, cols: (0, 0)),
                pl.BlockSpec((1, B), lambda i, rows, cols: (0, 0)),
            ],
            scratch_shapes=[pltpu.VMEM((B, CHUNK_W), jnp.float32)],
        ),
        out_shape=[
            jax.ShapeDtypeStruct((1, B), jnp.float32),
            jax.ShapeDtypeStruct((1, B), jnp.int32),
            jax.ShapeDtypeStruct((1, B), jnp.int32),
        ],
    )(
        rows.reshape(B),
        cols.reshape(B),
        logits_last,
        osel.reshape(B, 1),
        bflat.reshape(B, 1),
        bcol.reshape(B, 1),
    )

    return scores.reshape(B), prev.reshape(B), tok.reshape(B)


# speedup vs baseline: 1.2393x; 1.1861x over previous
import jax
import jax.numpy as jnp
from jax.experimental import pallas as pl
from jax.experimental.pallas import tpu as pltpu

B = 8
V = 1_000_000
BLK_W = 65_536
N_BLK = (V + BLK_W - 1) // BLK_W
CHUNK_W = 2_048
CH_PER_BLK = BLK_W // CHUNK_W
N_CHUNKS = N_BLK * CH_PER_BLK
NEG_INF = float("-inf")
BIG_I32 = 2**31 - 1


def _phase_a_body(x_ref, beam_ref, rows_ref, cols_ref, osel_ref, bflat_ref,
                  bcol_ref, cmax_ref, m_ref, s_ref):
    i = pl.program_id(0)

    @pl.when(i == 0)
    def _init():
        m_ref[...] = jnp.full((B, 1), NEG_INF, jnp.float32)
        s_ref[...] = jnp.zeros((B, 1), jnp.float32)

    def _update(x):
        tile = jnp.max(x.reshape(B, CH_PER_BLK, CHUNK_W), axis=2)
        cmax_ref[pl.ds(i, 1)] = tile[None]

        bm = jnp.max(tile, axis=1, keepdims=True)
        m_old = m_ref[...]
        m_new = jnp.maximum(m_old, bm)
        s_contrib = jnp.sum(jnp.exp(x - m_new), axis=1, keepdims=True)
        s_ref[...] = s_ref[...] * jnp.exp(m_old - m_new) + s_contrib
        m_ref[...] = m_new

    @pl.when(i < N_BLK - 1)
    def _fast():
        _update(x_ref[...])

    @pl.when(i == N_BLK - 1)
    def _fin():
        col = i * BLK_W + jax.lax.broadcasted_iota(jnp.int32, (B, BLK_W), 1)
        _update(jnp.where(col < V, x_ref[...], NEG_INF))

        lse = m_ref[...] + jnp.log(s_ref[...])
        o = beam_ref[...] - lse
        biased = cmax_ref[...] + o[None]
        sh = (N_BLK, B, CH_PER_BLK)
        d0 = jax.lax.broadcasted_iota(jnp.int32, sh, 0)
        d1 = jax.lax.broadcasted_iota(jnp.int32, sh, 1)
        d2 = jax.lax.broadcasted_iota(jnp.int32, sh, 2)
        key = d1 * N_CHUNKS + d0 * CH_PER_BLK + d2

        def _amax(v):
            return jnp.max(
                jnp.max(jnp.max(v, axis=2, keepdims=True), axis=1,
                        keepdims=True), axis=0, keepdims=True)

        def _amin(v):
            return jnp.min(
                jnp.min(jnp.min(v, axis=2, keepdims=True), axis=1,
                        keepdims=True), axis=0, keepdims=True)

        for k in range(B):
            m = _amax(biased)
            sel = _amin(jnp.where(biased == m, key, BIG_I32))
            r = sel // N_CHUNKS
            c = sel % N_CHUNKS
            rows_ref[0:1, k : k + 1] = r[0]
            cols_ref[0:1, k : k + 1] = c[0]
            osel_ref[0:1, k : k + 1] = _amax(
                jnp.where(d1 == r, o[None], NEG_INF))[0]
            bflat_ref[0:1, k : k + 1] = (r * V + c * CHUNK_W)[0]
            bcol_ref[0:1, k : k + 1] = (c * CHUNK_W)[0]
            biased = jnp.where(key == sel, NEG_INF, biased)


def _phase_b_body(rows_sref, cols_sref, blk_ref, osel_ref, bflat_ref,
                  bcol_ref, scores_ref, prev_ref, tok_ref, gath_ref):
    i = pl.program_id(0)
    r = rows_sref[i]
    gath_ref[pl.ds(i, 1), :] = blk_ref[pl.ds(r, 1), :]

    @pl.when(i == B - 1)
    def _fin():
        lane = jax.lax.broadcasted_iota(jnp.int32, (B, CHUNK_W), 1)
        colg = bcol_ref[...] + lane
        cand = jnp.where(colg < V, gath_ref[...] + osel_ref[...], NEG_INF)
        flat = bflat_ref[...] + lane

        def _amax(x):
            return jnp.max(
                jnp.max(x, axis=1, keepdims=True), axis=0, keepdims=True
            )

        def _amin(x):
            return jnp.min(
                jnp.min(x, axis=1, keepdims=True), axis=0, keepdims=True
            )

        for k in range(B):
            m = _amax(cand)
            sel = _amin(jnp.where(cand == m, flat, BIG_I32))
            scores_ref[0:1, k : k + 1] = m
            prev_ref[0:1, k : k + 1] = sel // V
            tok_ref[0:1, k : k + 1] = sel % V
            cand = jnp.where(flat == sel, NEG_INF, cand)


def kernel(logits_last, beam_scores):
    rows, cols, osel, bflat, bcol = pl.pallas_call(
        _phase_a_body,
        grid=(N_BLK,),
        in_specs=[
            pl.BlockSpec((B, BLK_W), lambda i: (0, i)),
            pl.BlockSpec((B, 1), lambda i: (0, 0)),
        ],
        out_specs=[
            pl.BlockSpec((1, B), lambda i: (0, 0)),
            pl.BlockSpec((1, B), lambda i: (0, 0)),
            pl.BlockSpec((1, B), lambda i: (0, 0)),
            pl.BlockSpec((1, B), lambda i: (0, 0)),
            pl.BlockSpec((1, B), lambda i: (0, 0)),
        ],
        out_shape=[
            jax.ShapeDtypeStruct((1, B), jnp.int32),
            jax.ShapeDtypeStruct((1, B), jnp.int32),
            jax.ShapeDtypeStruct((1, B), jnp.float32),
            jax.ShapeDtypeStruct((1, B), jnp.int32),
            jax.ShapeDtypeStruct((1, B), jnp.int32),
        ],
        scratch_shapes=[
            pltpu.VMEM((N_BLK, B, CH_PER_BLK), jnp.float32),
            pltpu.VMEM((B, 1), jnp.float32),
            pltpu.VMEM((B, 1), jnp.float32),
        ],
    )(logits_last, beam_scores.reshape(B, 1))

    scores, prev, tok = pl.pallas_call(
        _phase_b_body,
        grid_spec=pltpu.PrefetchScalarGridSpec(
            num_scalar_prefetch=2,
            grid=(B,),
            in_specs=[
                pl.BlockSpec(
                    (B, CHUNK_W), lambda i, rows, cols: (0, cols[i])
                ),
                pl.BlockSpec((B, 1), lambda i, rows, cols: (0, 0)),
                pl.BlockSpec((B, 1), lambda i, rows, cols: (0, 0)),
                pl.BlockSpec((B, 1), lambda i, rows, cols: (0, 0)),
            ],
            out_specs=[
                pl.BlockSpec((1, B), lambda i, rows, cols: (0, 0)),
                pl.BlockSpec((1, B), lambda i, rows, cols: (0, 0)),
                pl.BlockSpec((1, B), lambda i, rows, cols: (0, 0)),
            ],
            scratch_shapes=[pltpu.VMEM((B, CHUNK_W), jnp.float32)],
        ),
        out_shape=[
            jax.ShapeDtypeStruct((1, B), jnp.float32),
            jax.ShapeDtypeStruct((1, B), jnp.int32),
            jax.ShapeDtypeStruct((1, B), jnp.int32),
        ],
    )(
        rows.reshape(B),
        cols.reshape(B),
        logits_last,
        osel.reshape(B, 1),
        bflat.reshape(B, 1),
        bcol.reshape(B, 1),
    )

    return scores.reshape(B), prev.reshape(B), tok.reshape(B)


# speedup vs baseline: 1.3540x; 1.0926x over previous
import jax
import jax.numpy as jnp
from jax.experimental import pallas as pl
from jax.experimental.pallas import tpu as pltpu

B = 8
V = 1_000_000
BLK_W = 131_072
N_BLK = (V + BLK_W - 1) // BLK_W
CHUNK_W = 2_048
CH_PER_BLK = BLK_W // CHUNK_W
N_CHUNKS = N_BLK * CH_PER_BLK
NEG_INF = float("-inf")
BIG_I32 = 2**31 - 1


def _phase_a_body(x_ref, beam_ref, rows_ref, cols_ref, osel_ref, bflat_ref,
                  bcol_ref, cmax_ref, m_ref, s_ref):
    i = pl.program_id(0)

    @pl.when(i == 0)
    def _init():
        m_ref[...] = jnp.full((B, 1), NEG_INF, jnp.float32)
        s_ref[...] = jnp.zeros((B, 1), jnp.float32)

    def _update(x):
        tile = jnp.max(x.reshape(B, CH_PER_BLK, CHUNK_W), axis=2)
        cmax_ref[pl.ds(i, 1)] = tile[None]

        bm = jnp.max(tile, axis=1, keepdims=True)
        m_old = m_ref[...]
        m_new = jnp.maximum(m_old, bm)
        s_contrib = jnp.sum(jnp.exp(x - m_new), axis=1, keepdims=True)
        s_ref[...] = s_ref[...] * jnp.exp(m_old - m_new) + s_contrib
        m_ref[...] = m_new

    @pl.when(i < N_BLK - 1)
    def _fast():
        _update(x_ref[...])

    @pl.when(i == N_BLK - 1)
    def _fin():
        col = i * BLK_W + jax.lax.broadcasted_iota(jnp.int32, (B, BLK_W), 1)
        _update(jnp.where(col < V, x_ref[...], NEG_INF))

        lse = m_ref[...] + jnp.log(s_ref[...])
        o = beam_ref[...] - lse
        biased = cmax_ref[...] + o[None]
        sh = (N_BLK, B, CH_PER_BLK)
        d0 = jax.lax.broadcasted_iota(jnp.int32, sh, 0)
        d1 = jax.lax.broadcasted_iota(jnp.int32, sh, 1)
        d2 = jax.lax.broadcasted_iota(jnp.int32, sh, 2)
        key = d1 * N_CHUNKS + d0 * CH_PER_BLK + d2

        def _amax(v):
            return jnp.max(
                jnp.max(jnp.max(v, axis=2, keepdims=True), axis=1,
                        keepdims=True), axis=0, keepdims=True)

        def _amin(v):
            return jnp.min(
                jnp.min(jnp.min(v, axis=2, keepdims=True), axis=1,
                        keepdims=True), axis=0, keepdims=True)

        for k in range(B):
            m = _amax(biased)
            sel = _amin(jnp.where(biased == m, key, BIG_I32))
            r = sel // N_CHUNKS
            c = sel % N_CHUNKS
            rows_ref[0:1, k : k + 1] = r[0]
            cols_ref[0:1, k : k + 1] = c[0]
            osel_ref[0:1, k : k + 1] = _amax(
                jnp.where(d1 == r, o[None], NEG_INF))[0]
            bflat_ref[0:1, k : k + 1] = (r * V + c * CHUNK_W)[0]
            bcol_ref[0:1, k : k + 1] = (c * CHUNK_W)[0]
            biased = jnp.where(key == sel, NEG_INF, biased)


def _phase_b_body(rows_sref, cols_sref, blk_ref, osel_ref, bflat_ref,
                  bcol_ref, scores_ref, prev_ref, tok_ref, gath_ref):
    i = pl.program_id(0)
    r = rows_sref[i]
    gath_ref[pl.ds(i, 1), :] = blk_ref[pl.ds(r, 1), :]

    @pl.when(i == B - 1)
    def _fin():
        lane = jax.lax.broadcasted_iota(jnp.int32, (B, CHUNK_W), 1)
        colg = bcol_ref[...] + lane
        cand = jnp.where(colg < V, gath_ref[...] + osel_ref[...], NEG_INF)
        flat = bflat_ref[...] + lane

        def _amax(x):
            return jnp.max(
                jnp.max(x, axis=1, keepdims=True), axis=0, keepdims=True
            )

        def _amin(x):
            return jnp.min(
                jnp.min(x, axis=1, keepdims=True), axis=0, keepdims=True
            )

        for k in range(B):
            m = _amax(cand)
            sel = _amin(jnp.where(cand == m, flat, BIG_I32))
            scores_ref[0:1, k : k + 1] = m
            prev_ref[0:1, k : k + 1] = sel // V
            tok_ref[0:1, k : k + 1] = sel % V
            cand = jnp.where(flat == sel, NEG_INF, cand)


def kernel(logits_last, beam_scores):
    rows, cols, osel, bflat, bcol = pl.pallas_call(
        _phase_a_body,
        grid=(N_BLK,),
        in_specs=[
            pl.BlockSpec((B, BLK_W), lambda i: (0, i)),
            pl.BlockSpec((B, 1), lambda i: (0, 0)),
        ],
        out_specs=[
            pl.BlockSpec((1, B), lambda i: (0, 0)),
            pl.BlockSpec((1, B), lambda i: (0, 0)),
            pl.BlockSpec((1, B), lambda i: (0, 0)),
            pl.BlockSpec((1, B), lambda i: (0, 0)),
            pl.BlockSpec((1, B), lambda i: (0, 0)),
        ],
        out_shape=[
            jax.ShapeDtypeStruct((1, B), jnp.int32),
            jax.ShapeDtypeStruct((1, B), jnp.int32),
            jax.ShapeDtypeStruct((1, B), jnp.float32),
            jax.ShapeDtypeStruct((1, B), jnp.int32),
            jax.ShapeDtypeStruct((1, B), jnp.int32),
        ],
        scratch_shapes=[
            pltpu.VMEM((N_BLK, B, CH_PER_BLK), jnp.float32),
            pltpu.VMEM((B, 1), jnp.float32),
            pltpu.VMEM((B, 1), jnp.float32),
        ],
    )(logits_last, beam_scores.reshape(B, 1))

    scores, prev, tok = pl.pallas_call(
        _phase_b_body,
        grid_spec=pltpu.PrefetchScalarGridSpec(
            num_scalar_prefetch=2,
            grid=(B,),
            in_specs=[
                pl.BlockSpec(
                    (B, CHUNK_W), lambda i, rows, cols: (0, cols[i])
                ),
                pl.BlockSpec((B, 1), lambda i, rows, cols: (0, 0)),
                pl.BlockSpec((B, 1), lambda i, rows, cols: (0, 0)),
                pl.BlockSpec((B, 1), lambda i, rows, cols: (0, 0)),
            ],
            out_specs=[
                pl.BlockSpec((1, B), lambda i, rows, cols: (0, 0)),
                pl.BlockSpec((1, B), lambda i, rows, cols: (0, 0)),
                pl.BlockSpec((1, B), lambda i, rows, cols: (0, 0)),
            ],
            scratch_shapes=[pltpu.VMEM((B, CHUNK_W), jnp.float32)],
        ),
        out_shape=[
            jax.ShapeDtypeStruct((1, B), jnp.float32),
            jax.ShapeDtypeStruct((1, B), jnp.int32),
            jax.ShapeDtypeStruct((1, B), jnp.int32),
        ],
    )(
        rows.reshape(B),
        cols.reshape(B),
        logits_last,
        osel.reshape(B, 1),
        bflat.reshape(B, 1),
        bcol.reshape(B, 1),
    )

    return scores.reshape(B), prev.reshape(B), tok.reshape(B)
